# baseline (device time: 123875 ns/iter reference)
import jax
import jax.numpy as jnp
from jax import lax
from jax.experimental import pallas as pl
from jax.experimental.pallas import tpu as pltpu

N_DEV = 16
M_PER = 128


def kernel(x, w_mat):
    k_total, m_loc = x.shape
    k_loc, n = w_mat.shape

    def body(x_ref, w_ref, out_ref, comm_ref, send_sems, recv_sems):
        my = lax.axis_index("i")
        left = lax.rem(my + N_DEV - 1, N_DEV)
        right = lax.rem(my + 1, N_DEV)

        barrier_sem = pltpu.get_barrier_semaphore()
        for nbr in (left, right):
            pl.semaphore_signal(
                barrier_sem, inc=1,
                device_id=(nbr,), device_id_type=pl.DeviceIdType.MESH,
            )
        pl.semaphore_wait(barrier_sem, 2)

        def partial(c):
            xs = x_ref[pl.ds(c * M_PER, M_PER), :]
            return lax.dot_general(
                xs, w_ref[:, :], (((1,), (0,)), ((), ())),
                preferred_element_type=jnp.float32,
            )

        c0 = lax.rem(my + N_DEV - 1, N_DEV)
        comm_ref[0, :, :] = partial(c0).astype(jnp.bfloat16)

        for s in range(N_DEV - 1):
            rdma = pltpu.make_async_remote_copy(
                src_ref=comm_ref.at[s],
                dst_ref=comm_ref.at[s + 1],
                send_sem=send_sems.at[s],
                recv_sem=recv_sems.at[s],
                device_id=(right,),
                device_id_type=pl.DeviceIdType.MESH,
            )
            rdma.start()
            c = lax.rem(my + 2 * N_DEV - 2 - s, N_DEV)
            part = partial(c)
            rdma.wait()
            acc = part + comm_ref[s + 1, :, :].astype(jnp.float32)
            if s < N_DEV - 2:
                comm_ref[s + 1, :, :] = acc.astype(jnp.bfloat16)
            else:
                out_ref[:, :] = acc

    return pl.pallas_call(
        body,
        out_shape=jax.ShapeDtypeStruct((M_PER, n), jnp.float32),
        in_specs=[
            pl.BlockSpec(memory_space=pltpu.VMEM),
            pl.BlockSpec(memory_space=pltpu.VMEM),
        ],
        out_specs=pl.BlockSpec(memory_space=pltpu.VMEM),
        scratch_shapes=[
            pltpu.VMEM((N_DEV, M_PER, n), jnp.bfloat16),
            pltpu.SemaphoreType.DMA((N_DEV - 1,)),
            pltpu.SemaphoreType.DMA((N_DEV - 1,)),
        ],
        compiler_params=pltpu.CompilerParams(collective_id=0),
    )(x, w_mat)


# device time: 78801 ns/iter; 1.5720x vs baseline; 1.5720x over previous
import jax
import jax.numpy as jnp
from jax import lax
from jax.experimental import pallas as pl
from jax.experimental.pallas import tpu as pltpu

N_DEV = 16
M_PER = 128
SUBS = 2


def kernel(x, w_mat):
    k_total, m_loc = x.shape
    k_loc, n = w_mat.shape
    nh = n // 2
    ns = nh // SUBS

    def body(x_ref, w_ref, out_ref, commR, commL, sendR, recvR, sendL, recvL):
        my = lax.axis_index("i")
        left = lax.rem(my + N_DEV - 1, N_DEV)
        right = lax.rem(my + 1, N_DEV)

        barrier_sem = pltpu.get_barrier_semaphore()
        for nbr in (left, right):
            pl.semaphore_signal(
                barrier_sem, inc=1,
                device_id=(nbr,), device_id_type=pl.DeviceIdType.MESH,
            )
        pl.semaphore_wait(barrier_sem, 2)

        def partial(c, lo):
            xs = x_ref[pl.ds(c * M_PER, M_PER), :]
            return lax.dot_general(
                xs, w_ref[:, lo:lo + nh], (((1,), (0,)), ((), ())),
                preferred_element_type=jnp.float32,
            )

        def mkR(s, j):
            return pltpu.make_async_remote_copy(
                src_ref=commR.at[s, :, pl.ds(j * ns, ns)],
                dst_ref=commR.at[s + 1, :, pl.ds(j * ns, ns)],
                send_sem=sendR.at[s, j],
                recv_sem=recvR.at[s, j],
                device_id=(right,),
                device_id_type=pl.DeviceIdType.MESH,
            )

        def mkL(s, j):
            return pltpu.make_async_remote_copy(
                src_ref=commL.at[s, :, pl.ds(j * ns, ns)],
                dst_ref=commL.at[s + 1, :, pl.ds(j * ns, ns)],
                send_sem=sendL.at[s, j],
                recv_sem=recvL.at[s, j],
                device_id=(left,),
                device_id_type=pl.DeviceIdType.MESH,
            )

        commR[0, :, :] = partial(lax.rem(my + N_DEV - 1, N_DEV), 0).astype(jnp.bfloat16)
        commL[0, :, :] = partial(lax.rem(my + 1, N_DEV), nh).astype(jnp.bfloat16)
        for j in range(SUBS):
            mkR(0, j).start()
            mkL(0, j).start()

        for s in range(N_DEV - 1):
            pR = partial(lax.rem(my + 2 * N_DEV - 2 - s, N_DEV), 0)
            pL = partial(lax.rem(my + 2 + s, N_DEV), nh)
            for j in range(SUBS):
                sl = slice(j * ns, (j + 1) * ns)
                mkR(s, j).wait_recv()
                accR = pR[:, sl] + commR[s + 1, :, sl].astype(jnp.float32)
                mkL(s, j).wait_recv()
                accL = pL[:, sl] + commL[s + 1, :, sl].astype(jnp.float32)
                if s < N_DEV - 2:
                    commR[s + 1, :, sl] = accR.astype(jnp.bfloat16)
                    commL[s + 1, :, sl] = accL.astype(jnp.bfloat16)
                    mkR(s + 1, j).start()
                    mkL(s + 1, j).start()
                else:
                    out_ref[:, sl] = accR
                    out_ref[:, nh + j * ns:nh + (j + 1) * ns] = accL

        for s in range(N_DEV - 1):
            for j in range(SUBS):
                mkR(s, j).wait_send()
                mkL(s, j).wait_send()

    return pl.pallas_call(
        body,
        out_shape=jax.ShapeDtypeStruct((M_PER, n), jnp.float32),
        in_specs=[
            pl.BlockSpec(memory_space=pltpu.VMEM),
            pl.BlockSpec(memory_space=pltpu.VMEM),
        ],
        out_specs=pl.BlockSpec(memory_space=pltpu.VMEM),
        scratch_shapes=[
            pltpu.VMEM((N_DEV, M_PER, nh), jnp.bfloat16),
            pltpu.VMEM((N_DEV, M_PER, nh), jnp.bfloat16),
            pltpu.SemaphoreType.DMA((N_DEV - 1, SUBS)),
            pltpu.SemaphoreType.DMA((N_DEV - 1, SUBS)),
            pltpu.SemaphoreType.DMA((N_DEV - 1, SUBS)),
            pltpu.SemaphoreType.DMA((N_DEV - 1, SUBS)),
        ],
        compiler_params=pltpu.CompilerParams(collective_id=0),
    )(x, w_mat)


# device time: 69983 ns/iter; 1.7701x vs baseline; 1.1260x over previous
import jax
import jax.numpy as jnp
from jax import lax
from jax.experimental import pallas as pl
from jax.experimental.pallas import tpu as pltpu

N_DEV = 16
M_PER = 128
SUBS = 4


def kernel(x, w_mat):
    k_total, m_loc = x.shape
    k_loc, n = w_mat.shape
    nh = n // 2
    ns = nh // SUBS

    def body(x_ref, w_ref, out_ref, commR, commL, sendR, recvR, sendL, recvL):
        my = lax.axis_index("i")
        left = lax.rem(my + N_DEV - 1, N_DEV)
        right = lax.rem(my + 1, N_DEV)

        barrier_sem = pltpu.get_barrier_semaphore()
        for nbr in (left, right):
            pl.semaphore_signal(
                barrier_sem, inc=1,
                device_id=(nbr,), device_id_type=pl.DeviceIdType.MESH,
            )
        pl.semaphore_wait(barrier_sem, 2)

        def partial(c, lo):
            xs = x_ref[pl.ds(c * M_PER, M_PER), :]
            return lax.dot_general(
                xs, w_ref[:, lo:lo + nh], (((1,), (0,)), ((), ())),
                preferred_element_type=jnp.float32,
            )

        def mkR(s, j):
            return pltpu.make_async_remote_copy(
                src_ref=commR.at[s, :, pl.ds(j * ns, ns)],
                dst_ref=commR.at[s + 1, :, pl.ds(j * ns, ns)],
                send_sem=sendR.at[s, j],
                recv_sem=recvR.at[s, j],
                device_id=(right,),
                device_id_type=pl.DeviceIdType.MESH,
            )

        def mkL(s, j):
            return pltpu.make_async_remote_copy(
                src_ref=commL.at[s, :, pl.ds(j * ns, ns)],
                dst_ref=commL.at[s + 1, :, pl.ds(j * ns, ns)],
                send_sem=sendL.at[s, j],
                recv_sem=recvL.at[s, j],
                device_id=(left,),
                device_id_type=pl.DeviceIdType.MESH,
            )

        commR[0, :, :] = partial(lax.rem(my + N_DEV - 1, N_DEV), 0).astype(jnp.bfloat16)
        commL[0, :, :] = partial(lax.rem(my + 1, N_DEV), nh).astype(jnp.bfloat16)
        for j in range(SUBS):
            mkR(0, j).start()
            mkL(0, j).start()

        for s in range(N_DEV - 1):
            pR = partial(lax.rem(my + 2 * N_DEV - 2 - s, N_DEV), 0)
            pL = partial(lax.rem(my + 2 + s, N_DEV), nh)
            for j in range(SUBS):
                sl = slice(j * ns, (j + 1) * ns)
                mkR(s, j).wait_recv()
                accR = pR[:, sl] + commR[s + 1, :, sl].astype(jnp.float32)
                mkL(s, j).wait_recv()
                accL = pL[:, sl] + commL[s + 1, :, sl].astype(jnp.float32)
                if s < N_DEV - 2:
                    commR[s + 1, :, sl] = accR.astype(jnp.bfloat16)
                    commL[s + 1, :, sl] = accL.astype(jnp.bfloat16)
                    mkR(s + 1, j).start()
                    mkL(s + 1, j).start()
                else:
                    out_ref[:, sl] = accR
                    out_ref[:, nh + j * ns:nh + (j + 1) * ns] = accL

        for s in range(N_DEV - 1):
            for j in range(SUBS):
                mkR(s, j).wait_send()
                mkL(s, j).wait_send()

    return pl.pallas_call(
        body,
        out_shape=jax.ShapeDtypeStruct((M_PER, n), jnp.float32),
        in_specs=[
            pl.BlockSpec(memory_space=pltpu.VMEM),
            pl.BlockSpec(memory_space=pltpu.VMEM),
        ],
        out_specs=pl.BlockSpec(memory_space=pltpu.VMEM),
        scratch_shapes=[
            pltpu.VMEM((N_DEV, M_PER, nh), jnp.bfloat16),
            pltpu.VMEM((N_DEV, M_PER, nh), jnp.bfloat16),
            pltpu.SemaphoreType.DMA((N_DEV - 1, SUBS)),
            pltpu.SemaphoreType.DMA((N_DEV - 1, SUBS)),
            pltpu.SemaphoreType.DMA((N_DEV - 1, SUBS)),
            pltpu.SemaphoreType.DMA((N_DEV - 1, SUBS)),
        ],
        compiler_params=pltpu.CompilerParams(collective_id=0),
    )(x, w_mat)


# device time: 65197 ns/iter; 1.9000x vs baseline; 1.0734x over previous
import jax
import jax.numpy as jnp
from jax import lax
from jax.experimental import pallas as pl
from jax.experimental.pallas import tpu as pltpu

N_DEV = 16
M_PER = 128
SUBS = 4


def kernel(x, w_mat):
    k_total, m_loc = x.shape
    k_loc, n = w_mat.shape
    nh = n // 2
    ns = nh // SUBS

    def body(x_ref, w_ref, out_ref, commR, commL, sendR, recvR, sendL, recvL):
        my = lax.axis_index("i")
        left = lax.rem(my + N_DEV - 1, N_DEV)
        right = lax.rem(my + 1, N_DEV)

        barrier_sem = pltpu.get_barrier_semaphore()
        for nbr in (left, right):
            pl.semaphore_signal(
                barrier_sem, inc=1,
                device_id=(nbr,), device_id_type=pl.DeviceIdType.MESH,
            )
        pl.semaphore_wait(barrier_sem, 2)

        def partial(c, lo, dtype=jnp.bfloat16):
            xs = x_ref[pl.ds(c * M_PER, M_PER), :]
            r = lax.dot_general(
                xs, w_ref[:, lo:lo + nh], (((1,), (0,)), ((), ())),
                preferred_element_type=jnp.float32,
            )
            return r.astype(dtype)

        def mkR(s, j):
            return pltpu.make_async_remote_copy(
                src_ref=commR.at[s, :, pl.ds(j * ns, ns)],
                dst_ref=commR.at[s + 1, :, pl.ds(j * ns, ns)],
                send_sem=sendR.at[s, j],
                recv_sem=recvR.at[s, j],
                device_id=(right,),
                device_id_type=pl.DeviceIdType.MESH,
            )

        def mkL(s, j):
            return pltpu.make_async_remote_copy(
                src_ref=commL.at[s, :, pl.ds(j * ns, ns)],
                dst_ref=commL.at[s + 1, :, pl.ds(j * ns, ns)],
                send_sem=sendL.at[s, j],
                recv_sem=recvL.at[s, j],
                device_id=(left,),
                device_id_type=pl.DeviceIdType.MESH,
            )

        commR[0, :, :] = partial(lax.rem(my + N_DEV - 1, N_DEV), 0)
        commL[0, :, :] = partial(lax.rem(my + 1, N_DEV), nh)
        for j in range(SUBS):
            mkR(0, j).start()
            mkL(0, j).start()

        for s in range(N_DEV - 1):
            last = s == N_DEV - 2
            dt = jnp.float32 if last else jnp.bfloat16
            pR = partial(lax.rem(my + 2 * N_DEV - 2 - s, N_DEV), 0, dt)
            pL = partial(lax.rem(my + 2 + s, N_DEV), nh, dt)
            for j in range(SUBS):
                sl = slice(j * ns, (j + 1) * ns)
                mkR(s, j).wait_recv()
                if not last:
                    commR[s + 1, :, sl] = pR[:, sl] + commR[s + 1, :, sl]
                    mkR(s + 1, j).start()
                else:
                    out_ref[:, sl] = pR[:, sl] + commR[s + 1, :, sl].astype(jnp.float32)
                mkL(s, j).wait_recv()
                if not last:
                    commL[s + 1, :, sl] = pL[:, sl] + commL[s + 1, :, sl]
                    mkL(s + 1, j).start()
                else:
                    out_ref[:, nh + j * ns:nh + (j + 1) * ns] = (
                        pL[:, sl] + commL[s + 1, :, sl].astype(jnp.float32)
                    )

        for s in range(N_DEV - 1):
            for j in range(SUBS):
                mkR(s, j).wait_send()
                mkL(s, j).wait_send()

    return pl.pallas_call(
        body,
        out_shape=jax.ShapeDtypeStruct((M_PER, n), jnp.float32),
        in_specs=[
            pl.BlockSpec(memory_space=pltpu.VMEM),
            pl.BlockSpec(memory_space=pltpu.VMEM),
        ],
        out_specs=pl.BlockSpec(memory_space=pltpu.VMEM),
        scratch_shapes=[
            pltpu.VMEM((N_DEV, M_PER, nh), jnp.bfloat16),
            pltpu.VMEM((N_DEV, M_PER, nh), jnp.bfloat16),
            pltpu.SemaphoreType.DMA((N_DEV - 1, SUBS)),
            pltpu.SemaphoreType.DMA((N_DEV - 1, SUBS)),
            pltpu.SemaphoreType.DMA((N_DEV - 1, SUBS)),
            pltpu.SemaphoreType.DMA((N_DEV - 1, SUBS)),
        ],
        compiler_params=pltpu.CompilerParams(collective_id=0),
    )(x, w_mat)
